# baseline (speedup 1.0000x reference)
import functools

import jax
import jax.numpy as jnp
from jax.experimental import pallas as pl

_N = 65536
_J_IN = 144
_J_OUT = 118
_C_IN = _J_IN * 3
_C_OUT = _J_OUT * 3
_BLOCK = 2048


def _shuffle_body(cols_ref, x_ref, o_ref):
    cols = cols_ref[0]
    onehot = (jax.lax.broadcasted_iota(jnp.int32, (_C_IN, _C_OUT), 0)
              == cols[None, :]).astype(jnp.float32)
    o_ref[...] = jnp.dot(x_ref[...], onehot,
                         preferred_element_type=jnp.float32)


@jax.jit
def kernel(joints, joint_maps):
    x = joints.reshape(_N, _C_IN)
    jm = joint_maps.astype(jnp.int32)
    cols = (jnp.repeat(jm * 3, 3) + jnp.tile(jnp.arange(3, dtype=jnp.int32),
                                             _J_OUT)).reshape(1, _C_OUT)
    out = pl.pallas_call(
        _shuffle_body,
        grid=(_N // _BLOCK,),
        in_specs=[
            pl.BlockSpec((1, _C_OUT), lambda i: (0, 0)),
            pl.BlockSpec((_BLOCK, _C_IN), lambda i: (i, 0)),
        ],
        out_specs=pl.BlockSpec((_BLOCK, _C_OUT), lambda i: (i, 0)),
        out_shape=jax.ShapeDtypeStruct((_N, _C_OUT), jnp.float32),
    )(cols, x)
    return out.reshape(_N, _J_OUT, 3)
